# baseline (speedup 1.0000x reference)
import functools

import jax
import jax.numpy as jnp
from jax.experimental import pallas as pl
from jax.experimental.pallas import tpu as pltpu

D = 2048
NC = 3
BRANCHES = ('l5', 'l4', 'l3', 'l2', 'l1')

S_IN_ROWS = [D, 180, 84, 36, 12]
S_OUT_W = [183, 128, 128, 128, 128]


def _stage_pieces(params, li):
    out = []
    co = 0
    offs_prev = {}
    if li > 0:
        po = 0
        for br in BRANCHES:
            if len(params[0][br]) > li - 1:
                for e in range(NC):
                    offs_prev[(br, e)] = po
                    po += params[e][br][li - 1][0].shape[1]
    for br in BRANCHES:
        if len(params[0][br]) > li:
            for e in range(NC):
                W, b = params[e][br][li]
                ro = 0 if li == 0 else offs_prev[(br, e)]
                out.append((W, b, ro, co))
                co += W.shape[1]
    return out


def _mish(x):
    return x * jnp.tanh(jax.nn.softplus(x))


def _make_kernel(meta):
    n_per_stage = [len(m) for m in meta]

    def body(*refs):
        in_ref = refs[0]
        pos = 1
        stage_refs = []
        for n in n_per_stage:
            ws = refs[pos:pos + n]
            bs = refs[pos + n:pos + 2 * n]
            stage_refs.append((ws, bs))
            pos += 2 * n
        iws = refs[pos:pos + NC]
        ibs = refs[pos + NC:pos + 2 * NC]
        pos += 2 * NC
        out_ref = refs[pos]
        scr = refs[pos + 1:]
        w_s = scr[0:5]
        b_s = scr[5:10]

        @pl.when(pl.program_id(0) == 0)
        def _pack():
            for si in range(5):
                if si > 0:
                    w_s[si][...] = jnp.zeros_like(w_s[si])
                ws, bs = stage_refs[si]
                for (pi, (rin, wout, ro, co)) in enumerate(meta[si]):
                    w_s[si][ro:ro + rin, co:co + wout] = ws[pi][...]
                    b_s[si][0:1, co:co + wout] = bs[pi][...][None, :]
                if si > 0:
                    b_s[si][0:1, meta[si][-1][3] + meta[si][-1][1]:] = (
                        jnp.zeros((1, S_OUT_W[si] - meta[si][-1][3]
                                   - meta[si][-1][1]), jnp.float32))

        x = in_ref[:, :D]
        lc = in_ref[:, D:D + NC]
        h = _mish(jnp.dot(x, w_s[0][...],
                          preferred_element_type=jnp.float32) + b_s[0][...])
        finals = []
        for si in range(1, 5):
            finals.append(h[:, S_IN_ROWS[si]:S_IN_ROWS[si] + NC])
            h = _mish(jnp.dot(h[:, :S_IN_ROWS[si]], w_s[si][...],
                              preferred_element_type=jnp.float32) + b_s[si][...])
        x1v, x2v, x3v, x4v = finals
        x5v = h[:, 0:NC]
        xs = [x5v, x4v, x3v, x2v, x1v]
        o3 = None
        for k in range(5):
            row = jnp.concatenate([iws[e][k:k + 1, 0:1] for e in range(NC)],
                                  axis=1)
            term = xs[k] * row
            o3 = term if o3 is None else o3 + term
        o3 = o3 + jnp.concatenate([ibs[e][...][None, :] for e in range(NC)],
                                  axis=1)
        o3 = _mish(o3)
        m0, m1, m2 = lc[:, 0], lc[:, 1], lc[:, 2]
        c0 = (m0 >= m1) & (m0 >= m2)
        c1 = jnp.logical_and(jnp.logical_not(c0), m1 >= m2)
        out_ref[...] = jnp.where(c0, o3[:, 0], jnp.where(c1, o3[:, 1], o3[:, 2]))

    return body


@functools.partial(jax.jit, static_argnames=("interpret",))
def _run(inputs, params, interpret=False):
    B = inputs.shape[0]
    TB = 512
    operands = [inputs]
    meta = []
    for li in range(5):
        pieces = _stage_pieces(params, li)
        meta.append([(W.shape[0] if li else D, W.shape[1], ro, co)
                     for (W, b, ro, co) in pieces])
        operands += [W for (W, _, _, _) in pieces]
        operands += [b for (_, b, _, _) in pieces]
    operands += [p['integ'][0][0] for p in params]
    operands += [p['integ'][0][1] for p in params]

    def const_map(shape):
        nd = len(shape)
        return pl.BlockSpec(shape, lambda i, _nd=nd: (0,) * _nd)

    in_specs = ([pl.BlockSpec((TB, D + NC), lambda i: (i, 0))]
                + [const_map(op.shape) for op in operands[1:]])
    scratch = ([pltpu.VMEM((S_IN_ROWS[si], S_OUT_W[si]), jnp.float32)
                for si in range(5)]
               + [pltpu.VMEM((1, S_OUT_W[si]), jnp.float32) for si in range(5)])
    return pl.pallas_call(
        _make_kernel(meta),
        grid=(B // TB,),
        in_specs=in_specs,
        out_specs=pl.BlockSpec((TB,), lambda i: (i,)),
        out_shape=jax.ShapeDtypeStruct((B,), jnp.float32),
        scratch_shapes=scratch,
        interpret=interpret,
    )(*operands)


def kernel(inputs, params):
    return _run(inputs, params)


# speedup vs baseline: 3.2502x; 3.2502x over previous
import functools

import jax
import jax.numpy as jnp
from jax.experimental import pallas as pl
from jax.experimental.pallas import tpu as pltpu

D = 2048
NC = 3
BRANCHES = ('l5', 'l4', 'l3', 'l2', 'l1')

S_W = [183, 128, 128, 128, 128]
FIN = [180, 84, 36, 12, 0]
B_OFF = [0, 183, 311, 439, 567]
IWO, IBO, SMALLS = 695, 710, 720


def _stage_pieces(params, li):
    out, co = [], 0
    offs_prev, po = {}, 0
    if li > 0:
        for br in BRANCHES:
            if len(params[0][br]) > li - 1:
                for e in range(NC):
                    offs_prev[(br, e)] = po
                    po += params[e][br][li - 1][0].shape[1]
    for br in BRANCHES:
        if len(params[0][br]) > li:
            for e in range(NC):
                W, b = params[e][br][li]
                ro = 0 if li == 0 else offs_prev[(br, e)]
                out.append((W.T, b, ro, co))
                co += W.shape[1]
    return out


def _mish(x):
    return x * jnp.tanh(jax.nn.softplus(x))


def _make_kernel(meta):
    n_per_stage = [len(m) for m in meta]

    def body(*refs):
        in_ref = refs[0]
        sm = refs[1]
        pos = 2
        stage_w_refs = []
        for n in n_per_stage:
            stage_w_refs.append(refs[pos:pos + n])
            pos += n
        out_ref = refs[pos]
        w_s = refs[pos + 1:pos + 6]

        @pl.when(pl.program_id(0) == 0)
        def _pack():
            for si in range(5):
                if si > 0:
                    w_s[si][...] = jnp.zeros_like(w_s[si])
                for pi, (wout, rin, ro, co) in enumerate(meta[si]):
                    if si == 0:
                        w_s[0][co:co + wout, :] = stage_w_refs[0][pi][...]
                    else:
                        w_s[si][co:co + wout, ro:ro + rin] = (
                            stage_w_refs[si][pi][...])

        x = in_ref[0:D, :]
        lc = in_ref[D:D + NC, :]
        h = _mish(jax.lax.dot_general(
            w_s[0][...], x, (((1,), (0,)), ((), ())),
            preferred_element_type=jnp.float32) + sm[B_OFF[0]:B_OFF[0] + S_W[0], :])
        finals = []
        for si in range(1, 5):
            finals.append(h[FIN[si - 1]:FIN[si - 1] + NC, :])
            h = _mish(jax.lax.dot_general(
                w_s[si][:, :S_W[si - 1]], h, (((1,), (0,)), ((), ())),
                preferred_element_type=jnp.float32)
                + sm[B_OFF[si]:B_OFF[si] + S_W[si], :])
        x1v, x2v, x3v, x4v = finals
        x5v = h[0:NC, :]
        o3 = sm[IBO:IBO + NC, :]
        for k, xv in enumerate([x5v, x4v, x3v, x2v, x1v]):
            o3 = o3 + xv * sm[IWO + NC * k:IWO + NC * (k + 1), :]
        o3 = _mish(o3)
        m0, m1, m2 = lc[0:1, :], lc[1:2, :], lc[2:3, :]
        c0 = (m0 >= m1) & (m0 >= m2)
        c1 = jnp.logical_and(jnp.logical_not(c0), m1 >= m2)
        res = jnp.where(c0, o3[0:1, :], jnp.where(c1, o3[1:2, :], o3[2:3, :]))
        out_ref[...] = res[0, :]

    return body


@functools.partial(jax.jit, static_argnames=("interpret",))
def _run(inputs, params, interpret=False):
    B = inputs.shape[0]
    TB = 512
    xt = inputs.T

    operands = [xt]
    meta = []
    w_leaves = []
    small_parts = []
    for li in range(5):
        pieces = _stage_pieces(params, li)
        meta.append([(Wt.shape[0], Wt.shape[1], ro, co)
                     for (Wt, b, ro, co) in pieces])
        w_leaves += [Wt for (Wt, _, _, _) in pieces]
        bs = [b for (_, b, _, _) in pieces]
        used = sum(b.shape[0] for b in bs)
        small_parts += bs
        if used < S_W[li]:
            small_parts.append(jnp.zeros((S_W[li] - used,), jnp.float32))
    for k in range(5):
        for p in params:
            small_parts.append(p['integ'][0][0][k, 0:1])
    for p in params:
        small_parts.append(p['integ'][0][1])
    small_parts.append(jnp.zeros((SMALLS - IBO - NC,), jnp.float32))
    smalls = jnp.concatenate(small_parts)[:, None]
    operands.append(smalls)
    operands += w_leaves

    def const_map(shape):
        nd = len(shape)
        return pl.BlockSpec(shape, lambda i, _nd=nd: (0,) * _nd)

    in_specs = ([pl.BlockSpec((D + NC, TB), lambda i: (0, i)),
                 const_map(smalls.shape)]
                + [const_map(w.shape) for w in w_leaves])
    scratch = [pltpu.VMEM((S_W[0], D), jnp.float32)] + [
        pltpu.VMEM((S_W[si], S_W[si - 1]), jnp.float32) for si in range(1, 5)]
    return pl.pallas_call(
        _make_kernel(meta),
        grid=(B // TB,),
        in_specs=in_specs,
        out_specs=pl.BlockSpec((TB,), lambda i: (i,)),
        out_shape=jax.ShapeDtypeStruct((B,), jnp.float32),
        scratch_shapes=scratch,
        interpret=interpret,
    )(*operands)


def kernel(inputs, params):
    return _run(inputs, params)


# speedup vs baseline: 4.7947x; 1.4752x over previous
import functools

import jax
import jax.numpy as jnp
from jax.experimental import pallas as pl
from jax.experimental.pallas import tpu as pltpu

D = 2048
NC = 3
BRANCHES = ('l5', 'l4', 'l3', 'l2', 'l1')

S_W = [183, 128, 128, 128, 128]
FIN = [180, 84, 36, 12, 0]
B_OFF = [0, 183, 270, 309, 324]
IWO, IBO = 327, 342
SMALLS = 456


def _stage_pieces(params, li):
    out, co = [], 0
    offs_prev, po = {}, 0
    if li > 0:
        for br in BRANCHES:
            if len(params[0][br]) > li - 1:
                for e in range(NC):
                    offs_prev[(br, e)] = po
                    po += params[e][br][li - 1][0].shape[1]
    for br in BRANCHES:
        if len(params[0][br]) > li:
            for e in range(NC):
                W, b = params[e][br][li]
                ro = 0 if li == 0 else offs_prev[(br, e)]
                out.append((W.T, b, ro, co))
                co += W.shape[1]
    return out


def _mish(x):
    u = jnp.exp(jnp.minimum(x, 25.0))
    t = u * (u + 2.0)
    return x * (t / (t + 2.0))


def _make_kernel(meta):
    n_per_stage = [len(m) for m in meta]

    def body(*refs):
        in_ref = refs[0]
        sm = refs[1]
        pos = 2
        stage_w_refs = []
        for n in n_per_stage:
            stage_w_refs.append(refs[pos:pos + n])
            pos += n
        out_ref = refs[pos]
        w_s = refs[pos + 1:pos + 6]
        iw_s = refs[pos + 6]

        @pl.when(pl.program_id(0) == 0)
        def _pack():
            for si in range(5):
                if si > 0:
                    w_s[si][...] = jnp.zeros_like(w_s[si])
                for pi, (wout, rin, ro, co) in enumerate(meta[si]):
                    if si == 0:
                        w_s[0][co:co + wout, :] = stage_w_refs[0][pi][...]
                    else:
                        w_s[si][co:co + wout, ro:ro + rin] = (
                            stage_w_refs[si][pi][...])
            for k in range(5):
                for e in range(NC):
                    iw_s[NC * k + e:NC * k + e + 1, :] = (
                        sm[IWO + 5 * e + k:IWO + 5 * e + k + 1, :])

        x = in_ref[0:D, :]
        lc = in_ref[D:D + NC, :]
        h = _mish(jax.lax.dot_general(
            w_s[0][...], x, (((1,), (0,)), ((), ())),
            preferred_element_type=jnp.float32) + sm[B_OFF[0]:B_OFF[0] + S_W[0], :])
        finals = []
        for si in range(1, 5):
            finals.append(h[FIN[si - 1]:FIN[si - 1] + NC, :])
            h = _mish(jax.lax.dot_general(
                w_s[si][:, :S_W[si - 1]], h, (((1,), (0,)), ((), ())),
                preferred_element_type=jnp.float32)
                + sm[B_OFF[si]:B_OFF[si] + S_W[si], :])
        x1v, x2v, x3v, x4v = finals
        x5v = h[0:NC, :]
        o3 = sm[IBO:IBO + NC, :]
        for k, xv in enumerate([x5v, x4v, x3v, x2v, x1v]):
            o3 = o3 + xv * iw_s[NC * k:NC * (k + 1), :]
        o3 = _mish(o3)
        m0, m1, m2 = lc[0:1, :], lc[1:2, :], lc[2:3, :]
        c0 = (m0 >= m1) & (m0 >= m2)
        c1 = jnp.logical_and(jnp.logical_not(c0), m1 >= m2)
        res = jnp.where(c0, o3[0:1, :], jnp.where(c1, o3[1:2, :], o3[2:3, :]))
        out_ref[...] = res[0, :]

    return body


@functools.partial(jax.jit, static_argnames=("interpret",))
def _run(inputs, params, interpret=False):
    B = inputs.shape[0]
    TB = 512
    xt = inputs.T

    operands = [xt]
    meta = []
    w_leaves = []
    small_parts = []
    for li in range(5):
        pieces = _stage_pieces(params, li)
        meta.append([(Wt.shape[0], Wt.shape[1], ro, co)
                     for (Wt, b, ro, co) in pieces])
        w_leaves += [Wt for (Wt, _, _, _) in pieces]
        small_parts += [b for (_, b, _, _) in pieces]
    for p in params:
        small_parts.append(p['integ'][0][0].ravel())
    for p in params:
        small_parts.append(p['integ'][0][1])
    small_parts.append(jnp.zeros((SMALLS - IBO - NC,), jnp.float32))
    smalls = jnp.concatenate(small_parts)[:, None]
    operands.append(smalls)
    operands += w_leaves

    def const_map(shape):
        nd = len(shape)
        return pl.BlockSpec(shape, lambda i, _nd=nd: (0,) * _nd)

    in_specs = ([pl.BlockSpec((D + NC, TB), lambda i: (0, i)),
                 const_map(smalls.shape)]
                + [const_map(w.shape) for w in w_leaves])
    scratch = ([pltpu.VMEM((S_W[0], D), jnp.float32)]
               + [pltpu.VMEM((S_W[si], S_W[si - 1]), jnp.float32)
                  for si in range(1, 5)]
               + [pltpu.VMEM((16, 1), jnp.float32)])
    return pl.pallas_call(
        _make_kernel(meta),
        grid=(B // TB,),
        in_specs=in_specs,
        out_specs=pl.BlockSpec((TB,), lambda i: (i,)),
        out_shape=jax.ShapeDtypeStruct((B,), jnp.float32),
        scratch_shapes=scratch,
        interpret=interpret,
    )(*operands)


def kernel(inputs, params):
    return _run(inputs, params)


# speedup vs baseline: 4.8035x; 1.0018x over previous
import functools

import jax
import jax.numpy as jnp
from jax.experimental import pallas as pl
from jax.experimental.pallas import tpu as pltpu

D = 2048
NC = 3
BRANCHES = ('l5', 'l4', 'l3', 'l2', 'l1')

S_W = [183, 128, 128, 128, 128]
FIN = [180, 84, 36, 12, 0]
B_OFF = [0, 183, 270, 309, 324]
IWO, IBO = 327, 342
SMALLS = 456


def _stage_pieces(params, li):
    out, co = [], 0
    offs_prev, po = {}, 0
    if li > 0:
        for br in BRANCHES:
            if len(params[0][br]) > li - 1:
                for e in range(NC):
                    offs_prev[(br, e)] = po
                    po += params[e][br][li - 1][0].shape[1]
    for br in BRANCHES:
        if len(params[0][br]) > li:
            for e in range(NC):
                W, b = params[e][br][li]
                ro = 0 if li == 0 else offs_prev[(br, e)]
                out.append((W.T, b, ro, co))
                co += W.shape[1]
    return out


def _mish(x):
    u = jnp.exp(jnp.minimum(x, 25.0))
    t = u * (u + 2.0)
    return x * (t / (t + 2.0))


def _make_kernel(meta):
    n_per_stage = [len(m) for m in meta]

    def body(*refs):
        in_ref = refs[0]
        sm = refs[1]
        pos = 2
        stage_w_refs = []
        for n in n_per_stage:
            stage_w_refs.append(refs[pos:pos + n])
            pos += n
        out_ref = refs[pos]
        w_s = refs[pos + 1:pos + 6]
        iw_s = refs[pos + 6]

        @pl.when(pl.program_id(0) == 0)
        def _pack():
            for si in range(5):
                if si > 0:
                    w_s[si][...] = jnp.zeros_like(w_s[si])
                for pi, (wout, rin, ro, co) in enumerate(meta[si]):
                    if si == 0:
                        w_s[0][co:co + wout, :] = stage_w_refs[0][pi][...]
                    else:
                        w_s[si][co:co + wout, ro:ro + rin] = (
                            stage_w_refs[si][pi][...])
            for k in range(5):
                for e in range(NC):
                    iw_s[NC * k + e:NC * k + e + 1, :] = (
                        sm[IWO + 5 * e + k:IWO + 5 * e + k + 1, :])

        x = in_ref[0:D, :]
        lc = in_ref[D:D + NC, :]
        h = _mish(jax.lax.dot_general(
            w_s[0][...], x, (((1,), (0,)), ((), ())),
            preferred_element_type=jnp.float32) + sm[B_OFF[0]:B_OFF[0] + S_W[0], :])
        finals = []
        for si in range(1, 5):
            finals.append(h[FIN[si - 1]:FIN[si - 1] + NC, :])
            h = _mish(jax.lax.dot_general(
                w_s[si][:, :S_W[si - 1]], h, (((1,), (0,)), ((), ())),
                preferred_element_type=jnp.float32)
                + sm[B_OFF[si]:B_OFF[si] + S_W[si], :])
        x1v, x2v, x3v, x4v = finals
        x5v = h[0:NC, :]
        o3 = sm[IBO:IBO + NC, :]
        for k, xv in enumerate([x5v, x4v, x3v, x2v, x1v]):
            o3 = o3 + xv * iw_s[NC * k:NC * (k + 1), :]
        o3 = _mish(o3)
        m0, m1, m2 = lc[0:1, :], lc[1:2, :], lc[2:3, :]
        c0 = (m0 >= m1) & (m0 >= m2)
        c1 = jnp.logical_and(jnp.logical_not(c0), m1 >= m2)
        res = jnp.where(c0, o3[0:1, :], jnp.where(c1, o3[1:2, :], o3[2:3, :]))
        out_ref[...] = res[0, :]

    return body


@functools.partial(jax.jit, static_argnames=("interpret", "tb"))
def _run(inputs, params, interpret=False, tb=512):
    B = inputs.shape[0]
    xt = inputs.T

    meta = []
    w_leaves = []
    small_parts = []
    for li in range(5):
        pieces = _stage_pieces(params, li)
        meta.append([(Wt.shape[0], Wt.shape[1], ro, co)
                     for (Wt, b, ro, co) in pieces])
        w_leaves += [Wt for (Wt, _, _, _) in pieces]
        small_parts += [b for (_, b, _, _) in pieces]
    for p in params:
        small_parts.append(p['integ'][0][0].ravel())
    for p in params:
        small_parts.append(p['integ'][0][1])
    small_parts.append(jnp.zeros((SMALLS - IBO - NC,), jnp.float32))
    smalls = jnp.concatenate(small_parts)[:, None]

    def const_map(shape):
        nd = len(shape)
        return pl.BlockSpec(shape, lambda i, _nd=nd: (0,) * _nd)

    in_specs = ([pl.BlockSpec((D + NC, tb), lambda i: (0, i)),
                 const_map(smalls.shape)]
                + [const_map(w.shape) for w in w_leaves])
    scratch = ([pltpu.VMEM((S_W[0], D), jnp.float32)]
               + [pltpu.VMEM((S_W[si], S_W[si - 1]), jnp.float32)
                  for si in range(1, 5)]
               + [pltpu.VMEM((16, 1), jnp.float32)])
    return pl.pallas_call(
        _make_kernel(meta),
        grid=(B // tb,),
        in_specs=in_specs,
        out_specs=pl.BlockSpec((tb,), lambda i: (i,)),
        out_shape=jax.ShapeDtypeStruct((B,), jnp.float32),
        scratch_shapes=scratch,
        interpret=interpret,
    )(xt, smalls, *w_leaves)


def kernel(inputs, params):
    return _run(inputs, params)


# speedup vs baseline: 5.3778x; 1.1196x over previous
import functools

import jax
import jax.numpy as jnp
from jax.experimental import pallas as pl
from jax.experimental.pallas import tpu as pltpu

D = 2048
NC = 3
BRANCHES = ('l5', 'l4', 'l3', 'l2', 'l1')

S_W = [183, 128, 128, 128, 128]
FIN = [180, 84, 36, 12, 0]
B_OFF = [0, 183, 270, 309, 324]
IWO, IBO = 327, 342
SMALLS = 456


def _stage_pieces(params, li):
    out, co = [], 0
    offs_prev, po = {}, 0
    if li > 0:
        for br in BRANCHES:
            if len(params[0][br]) > li - 1:
                for e in range(NC):
                    offs_prev[(br, e)] = po
                    po += params[e][br][li - 1][0].shape[1]
    for br in BRANCHES:
        if len(params[0][br]) > li:
            for e in range(NC):
                W, b = params[e][br][li]
                ro = 0 if li == 0 else offs_prev[(br, e)]
                out.append((W.T, b, ro, co))
                co += W.shape[1]
    return out


def _mish(x):
    u = jnp.exp(jnp.minimum(x, 25.0))
    t = u * (u + 2.0)
    return x * (t / (t + 2.0))


def _make_kernel(meta):
    n_per_stage = [len(m) for m in meta]

    def body(*refs):
        in_ref = refs[0]
        sm = refs[1]
        pos = 2
        stage_w_refs = []
        for n in n_per_stage:
            stage_w_refs.append(refs[pos:pos + n])
            pos += n
        out_ref = refs[pos]
        w_s = refs[pos + 1:pos + 6]
        iw_s = refs[pos + 6]

        @pl.when(pl.program_id(0) == 0)
        def _pack():
            for si in range(5):
                if si > 0:
                    w_s[si][...] = jnp.zeros_like(w_s[si])
                for pi, (wout, rin, ro, co) in enumerate(meta[si]):
                    if si == 0:
                        w_s[0][co:co + wout, :] = stage_w_refs[0][pi][...]
                    else:
                        w_s[si][co:co + wout, ro:ro + rin] = (
                            stage_w_refs[si][pi][...])
            for k in range(5):
                for e in range(NC):
                    iw_s[NC * k + e:NC * k + e + 1, :] = (
                        sm[IWO + 5 * e + k:IWO + 5 * e + k + 1, :])

        x = in_ref[0:D, :]
        lc = in_ref[D:D + NC, :]
        h = _mish(jax.lax.dot_general(
            w_s[0][...], x, (((1,), (0,)), ((), ())),
            preferred_element_type=jnp.float32) + sm[B_OFF[0]:B_OFF[0] + S_W[0], :])
        finals = []
        for si in range(1, 5):
            finals.append(h[FIN[si - 1]:FIN[si - 1] + NC, :])
            h = _mish(jax.lax.dot_general(
                w_s[si][:, :S_W[si - 1]], h, (((1,), (0,)), ((), ())),
                preferred_element_type=jnp.float32)
                + sm[B_OFF[si]:B_OFF[si] + S_W[si], :])
        x1v, x2v, x3v, x4v = finals
        x5v = h[0:NC, :]
        o3 = sm[IBO:IBO + NC, :]
        for k, xv in enumerate([x5v, x4v, x3v, x2v, x1v]):
            o3 = o3 + xv * iw_s[NC * k:NC * (k + 1), :]
        o3 = _mish(o3)
        m0, m1, m2 = lc[0:1, :], lc[1:2, :], lc[2:3, :]
        c0 = (m0 >= m1) & (m0 >= m2)
        c1 = jnp.logical_and(jnp.logical_not(c0), m1 >= m2)
        res = jnp.where(c0, o3[0:1, :], jnp.where(c1, o3[1:2, :], o3[2:3, :]))
        out_ref[...] = res[0, :]

    return body


@functools.partial(jax.jit, static_argnames=("interpret", "tb"))
def _run(inputs, params, interpret=False, tb=512):
    B = inputs.shape[0]
    xt = inputs.T

    meta = []
    w_leaves = []
    small_parts = []
    for li in range(5):
        pieces = _stage_pieces(params, li)
        meta.append([(Wt.shape[0], Wt.shape[1], ro, co)
                     for (Wt, b, ro, co) in pieces])
        w_leaves += [Wt for (Wt, _, _, _) in pieces]
        small_parts += [b for (_, b, _, _) in pieces]
    for p in params:
        small_parts.append(p['integ'][0][0].ravel())
    for p in params:
        small_parts.append(p['integ'][0][1])
    small_parts.append(jnp.zeros((SMALLS - IBO - NC,), jnp.float32))
    smalls = jnp.concatenate(small_parts)[:, None]

    def const_map(shape):
        nd = len(shape)
        return pl.BlockSpec(shape, lambda i, _nd=nd: (0,) * _nd)

    in_specs = ([pl.BlockSpec((D + NC, tb), lambda i: (0, i)),
                 const_map(smalls.shape)]
                + [const_map(w.shape) for w in w_leaves])
    scratch = ([pltpu.VMEM((S_W[0], D), jnp.float32)]
               + [pltpu.VMEM((S_W[si], S_W[si - 1]), jnp.float32)
                  for si in range(1, 5)]
               + [pltpu.VMEM((16, 1), jnp.float32)])
    return pl.pallas_call(
        _make_kernel(meta),
        grid=(B // tb,),
        in_specs=in_specs,
        out_specs=pl.BlockSpec((tb,), lambda i: (i,)),
        out_shape=jax.ShapeDtypeStruct((B,), jnp.float32),
        scratch_shapes=scratch,
        interpret=interpret,
    )(xt, smalls, *w_leaves)


def kernel(inputs, params):
    return _run(inputs, params, tb=1024)
